# baseline (speedup 1.0000x reference)
import jax
import jax.numpy as jnp
from jax.experimental import pallas as pl
from jax.experimental.pallas import tpu as pltpu

_NT = 32768
_H = 768
_NE = 64
_BT = 4096


def _body(x_ref, w_ref, w1_ref, w2_ref, i1_ref, i2_ref):
    logits = jax.lax.dot_general(
        w_ref[...], x_ref[...],
        dimension_numbers=(((1,), (1,)), ((), ())),
        preferred_element_type=jnp.float32)
    e_ids = jax.lax.broadcasted_iota(jnp.int32, logits.shape, 0)
    m1 = jnp.max(logits, axis=0, keepdims=True)
    i1 = jnp.min(jnp.where(logits == m1, e_ids, _NE), axis=0, keepdims=True)
    masked = jnp.where(e_ids == i1, -jnp.inf, logits)
    m2 = jnp.max(masked, axis=0, keepdims=True)
    i2 = jnp.min(jnp.where(masked == m2, e_ids, _NE), axis=0, keepdims=True)
    t = jnp.exp(m2 - m1)
    d = 1.0 + t
    w1_ref[...] = 1.0 / d
    w2_ref[...] = t / d
    i1_ref[...] = i1
    i2_ref[...] = i2


def kernel(x, W):
    w1, w2, i1, i2 = pl.pallas_call(
        _body,
        grid=(_NT // _BT,),
        in_specs=[
            pl.BlockSpec((_BT, _H), lambda i: (i, 0)),
            pl.BlockSpec((_NE, _H), lambda i: (0, 0)),
        ],
        out_specs=[
            pl.BlockSpec((1, _BT), lambda i: (0, i)),
            pl.BlockSpec((1, _BT), lambda i: (0, i)),
            pl.BlockSpec((1, _BT), lambda i: (0, i)),
            pl.BlockSpec((1, _BT), lambda i: (0, i)),
        ],
        out_shape=[
            jax.ShapeDtypeStruct((1, _NT), jnp.float32),
            jax.ShapeDtypeStruct((1, _NT), jnp.float32),
            jax.ShapeDtypeStruct((1, _NT), jnp.int32),
            jax.ShapeDtypeStruct((1, _NT), jnp.int32),
        ],
        compiler_params=pltpu.CompilerParams(
            dimension_semantics=("arbitrary",)),
    )(x, W)
    rw = jnp.stack([w1[0], w2[0]], axis=-1)
    se = jnp.stack([i1[0], i2[0]], axis=-1)
    return (rw, se)


# speedup vs baseline: 1.0971x; 1.0971x over previous
import jax
import jax.numpy as jnp
from jax.experimental import pallas as pl
from jax.experimental.pallas import tpu as pltpu

_NT = 32768
_H = 768
_NE = 64
_BT = 4096


def _body(x_ref, w_ref, rw_ref, se_ref):
    logits = jax.lax.dot_general(
        w_ref[...], x_ref[...],
        dimension_numbers=(((1,), (1,)), ((), ())),
        preferred_element_type=jnp.float32)
    e_ids = jax.lax.broadcasted_iota(jnp.int32, logits.shape, 0)
    m1 = jnp.max(logits, axis=0, keepdims=True)
    i1 = jnp.min(jnp.where(logits == m1, e_ids, _NE), axis=0, keepdims=True)
    masked = jnp.where(e_ids == i1, -jnp.inf, logits)
    m2 = jnp.max(masked, axis=0, keepdims=True)
    i2 = jnp.min(jnp.where(masked == m2, e_ids, _NE), axis=0, keepdims=True)
    t = jnp.exp(m2 - m1)
    d = 1.0 + t
    rw_ref[...] = jnp.concatenate([1.0 / d, t / d], axis=0)
    se_ref[...] = jnp.concatenate([i1, i2], axis=0)


def kernel(x, W):
    rw_t, se_t = pl.pallas_call(
        _body,
        grid=(_NT // _BT,),
        in_specs=[
            pl.BlockSpec((_BT, _H), lambda i: (i, 0)),
            pl.BlockSpec((_NE, _H), lambda i: (0, 0)),
        ],
        out_specs=[
            pl.BlockSpec((2, _BT), lambda i: (0, i)),
            pl.BlockSpec((2, _BT), lambda i: (0, i)),
        ],
        out_shape=[
            jax.ShapeDtypeStruct((2, _NT), jnp.float32),
            jax.ShapeDtypeStruct((2, _NT), jnp.int32),
        ],
        compiler_params=pltpu.CompilerParams(
            dimension_semantics=("arbitrary",)),
    )(x, W)
    return (rw_t.T, se_t.T)
